# Initial kernel scaffold; baseline (speedup 1.0000x reference)
import jax
import jax.numpy as jnp
from jax import lax
from jax.experimental import pallas as pl
from jax.experimental.pallas import tpu as pltpu
from jax.experimental.pallas import tpu_sc as plsc

NN = 50000
NPAD = 51200
EPAD = 851968
CHUNK = 128
TILE_ROWS = NPAD // 16
NB = 8
BLK = NPAD // NB
NEG = -3.0e38
HP = lax.Precision.HIGHEST


def _dot(a, b, dims):
    return lax.dot_general(a, b, (dims, ((), ())),
                           preferred_element_type=jnp.float32, precision=HP)


def _a1_body(x_ref, w1_ref, asrc_ref, adst_ref, as_ref, ad_ref, g_ref, m_sc):
    i = pl.program_id(0)
    ab = jnp.concatenate([asrc_ref[...], adst_ref[...]], axis=0)
    cv = _dot(w1_ref[...], ab, ((1,), (1,)))
    xb = x_ref[...][:, 0:3]
    sd = _dot(xb, cv, ((1,), (0,)))
    as_ref[...] = sd[:, 0:1]
    ad_ref[...] = sd[:, 1:2]
    rows = i * BLK + lax.broadcasted_iota(jnp.int32, (BLK, 1), 0)
    bm = jnp.max(jnp.where(rows < NN, sd[:, 0:1], NEG))

    @pl.when(i == 0)
    def _():
        m_sc[0, 0] = NEG

    m_sc[0, 0] = jnp.maximum(m_sc[0, 0], bm)
    g_ref[...] = jnp.full((1, 128), m_sc[0, 0], jnp.float32)


def _a1_call(x16, W1, a_src1, a_dst1):
    return pl.pallas_call(
        _a1_body,
        grid=(NB,),
        in_specs=[
            pl.BlockSpec((BLK, 16), lambda i: (i, 0)),
            pl.BlockSpec((3, 64), lambda i: (0, 0)),
            pl.BlockSpec((1, 64), lambda i: (0, 0)),
            pl.BlockSpec((1, 64), lambda i: (0, 0)),
        ],
        out_specs=[
            pl.BlockSpec((BLK, 1), lambda i: (i, 0)),
            pl.BlockSpec((BLK, 1), lambda i: (i, 0)),
            pl.BlockSpec((1, 128), lambda i: (0, 0)),
        ],
        out_shape=[
            jax.ShapeDtypeStruct((NPAD, 1), jnp.float32),
            jax.ShapeDtypeStruct((NPAD, 1), jnp.float32),
            jax.ShapeDtypeStruct((1, 128), jnp.float32),
        ],
        scratch_shapes=[pltpu.SMEM((1, 1), jnp.float32)],
    )(x16, W1, a_src1, a_dst1)


def _b1_body(acc_ref, w1_ref, b1_ref, w2_ref, asrc_ref, adst_ref,
             h2_ref, as_ref, ad_ref, g_ref, m_sc):
    i = pl.program_id(0)
    a = acc_ref[...]
    agg = a[0] + a[1]
    xw = agg[:, 0:3] / (agg[:, 3:4] + 1e-16)
    h1 = jnp.maximum(_dot(xw, w1_ref[...], ((1,), (0,))) + b1_ref[...], 0.0)
    h2 = _dot(h1, w2_ref[...], ((1,), (0,)))
    h2_ref[...] = h2
    ab = jnp.concatenate([asrc_ref[...], adst_ref[...]], axis=0)
    sd = _dot(h2, ab, ((1,), (1,)))
    as_ref[...] = sd[:, 0:1]
    ad_ref[...] = sd[:, 1:2]
    rows = i * BLK + lax.broadcasted_iota(jnp.int32, (BLK, 1), 0)
    bm = jnp.max(jnp.where(rows < NN, sd[:, 0:1], NEG))

    @pl.when(i == 0)
    def _():
        m_sc[0, 0] = NEG

    m_sc[0, 0] = jnp.maximum(m_sc[0, 0], bm)
    g_ref[...] = jnp.full((1, 128), m_sc[0, 0], jnp.float32)


def _b1_call(acc1, W1, b1, W2, a_src2, a_dst2):
    return pl.pallas_call(
        _b1_body,
        grid=(NB,),
        in_specs=[
            pl.BlockSpec((2, BLK, 16), lambda i: (0, i, 0)),
            pl.BlockSpec((3, 64), lambda i: (0, 0)),
            pl.BlockSpec((1, 64), lambda i: (0, 0)),
            pl.BlockSpec((64, 64), lambda i: (0, 0)),
            pl.BlockSpec((1, 64), lambda i: (0, 0)),
            pl.BlockSpec((1, 64), lambda i: (0, 0)),
        ],
        out_specs=[
            pl.BlockSpec((BLK, 64), lambda i: (i, 0)),
            pl.BlockSpec((BLK, 1), lambda i: (i, 0)),
            pl.BlockSpec((BLK, 1), lambda i: (i, 0)),
            pl.BlockSpec((1, 128), lambda i: (0, 0)),
        ],
        out_shape=[
            jax.ShapeDtypeStruct((NPAD, 64), jnp.float32),
            jax.ShapeDtypeStruct((NPAD, 1), jnp.float32),
            jax.ShapeDtypeStruct((NPAD, 1), jnp.float32),
            jax.ShapeDtypeStruct((1, 128), jnp.float32),
        ],
        scratch_shapes=[pltpu.SMEM((1, 1), jnp.float32)],
    )(acc1, W1, b1, W2, a_src2, a_dst2)


def _c_body(acc_ref, dn_ref, b2_ref, batch_ref, wh1_ref, bh1_ref,
            wh2_ref, bh2_ref, out_ref, s_sc):
    i = pl.program_id(0)
    a = acc_ref[...]
    h = jnp.concatenate([a[0], a[1]], axis=1)
    h = jnp.maximum(h / (dn_ref[...] + 1e-16) + b2_ref[...], 0.0)
    oh = (batch_ref[...] == lax.broadcasted_iota(jnp.int32, (1, 64), 1))
    oh = oh.astype(jnp.float32)
    hx = jnp.concatenate([h, jnp.ones((BLK, 64), jnp.float32)], axis=1)
    ps = _dot(oh, hx, ((0,), (0,)))

    @pl.when(i == 0)
    def _():
        s_sc[...] = jnp.zeros((64, 128), jnp.float32)

    s_sc[...] += ps

    @pl.when(i == NB - 1)
    def _():
        sums = s_sc[:, 0:64]
        cnt = jnp.maximum(s_sc[:, 64:65], 1.0)
        pooled = sums / cnt
        t = jnp.maximum(_dot(pooled, wh1_ref[...], ((1,), (0,))) + bh1_ref[...],
                        0.0)
        out_ref[...] = _dot(t, wh2_ref[...], ((1,), (0,))) + bh2_ref[...]

    @pl.when(i < NB - 1)
    def _():
        out_ref[...] = jnp.zeros((64, 1), jnp.float32)


def _c_call(acc2, dn, b2, batchp, Wh1, bh1, Wh2, bh2):
    return pl.pallas_call(
        _c_body,
        grid=(NB,),
        in_specs=[
            pl.BlockSpec((2, BLK, 32), lambda i: (0, i, 0)),
            pl.BlockSpec((BLK, 1), lambda i: (i, 0)),
            pl.BlockSpec((1, 64), lambda i: (0, 0)),
            pl.BlockSpec((BLK, 1), lambda i: (i, 0)),
            pl.BlockSpec((64, 128), lambda i: (0, 0)),
            pl.BlockSpec((1, 128), lambda i: (0, 0)),
            pl.BlockSpec((128, 1), lambda i: (0, 0)),
            pl.BlockSpec((1, 1), lambda i: (0, 0)),
        ],
        out_specs=pl.BlockSpec((64, 1), lambda i: (0, 0)),
        out_shape=jax.ShapeDtypeStruct((64, 1), jnp.float32),
        scratch_shapes=[pltpu.VMEM((64, 128), jnp.float32)],
    )(acc2, dn, b2, batchp, Wh1, bh1, Wh2, bh2)


def _sc_layer(D, split_edges, do_den, double_idx):
    mesh = plsc.VectorSubcoreMesh(core_axis_name="c", subcore_axis_name="s",
                                  num_cores=2, num_subcores=16)
    out_type = [jax.ShapeDtypeStruct((2, NPAD, D), jnp.float32)]
    if do_den:
        out_type.append(jax.ShapeDtypeStruct((NPAD,), jnp.float32))
    scratch = [
        pltpu.VMEM_SHARED((NPAD, D), jnp.float32),
        pltpu.VMEM_SHARED((NPAD,), jnp.float32),
        pltpu.VMEM((CHUNK,), jnp.int32),
        pltpu.VMEM((CHUNK,), jnp.int32),
        pltpu.VMEM((CHUNK,), jnp.int32),
        pltpu.VMEM((CHUNK,), jnp.float32),
        pltpu.VMEM((CHUNK,), jnp.float32),
        pltpu.VMEM((CHUNK,), jnp.float32),
        pltpu.VMEM((CHUNK, D), jnp.float32),
        pltpu.VMEM((128,), jnp.float32),
        pltpu.SemaphoreType.DMA,
        pltpu.SemaphoreType.DMA,
        pltpu.SemaphoreType.DMA,
    ]

    if split_edges:
        n_chunks = EPAD // 32 // CHUNK
    else:
        n_chunks = EPAD // 16 // CHUNK

    def body(src_hbm, dst_hbm, as_hbm, ad_hbm, g_hbm, tab_hbm, z_hbm, zd_hbm,
             *refs):
        if do_den:
            acc_out, den_out = refs[0], refs[1]
            rest = refs[2:]
        else:
            acc_out = refs[0]
            rest = refs[1:]
        (acc_sh, den_sh, srcb, dstb, idxb, sb, db, wb, rows, gb,
         sem_s, sem_d, sem_r) = rest
        c = lax.axis_index("c")
        s = lax.axis_index("s")
        base_rows = s * TILE_ROWS
        pltpu.sync_copy(z_hbm.at[pl.ds(base_rows, TILE_ROWS)],
                        acc_sh.at[pl.ds(base_rows, TILE_ROWS)])
        if do_den:
            @pl.when(c == 0)
            def _():
                pltpu.sync_copy(zd_hbm.at[pl.ds(base_rows, TILE_ROWS)],
                                den_sh.at[pl.ds(base_rows, TILE_ROWS)])
        pltpu.sync_copy(g_hbm, gb)
        plsc.subcore_barrier()
        g = gb[0]
        if split_edges:
            tile_base = c * (EPAD // 2) + s * (EPAD // 32)
        else:
            tile_base = s * (EPAD // 16)

        def chunk_body(i, carry):
            base = tile_base + i * CHUNK
            pltpu.sync_copy(src_hbm.at[pl.ds(base, CHUNK)], srcb)
            pltpu.sync_copy(dst_hbm.at[pl.ds(base, CHUNK)], dstb)
            cp_s = pltpu.async_copy(as_hbm.at[srcb], sb, sem_s)
            cp_d = pltpu.async_copy(ad_hbm.at[dstb], db, sem_d)
            if double_idx:
                for j in range(CHUNK // 16):
                    v = srcb[pl.ds(j * 16, 16)]
                    idxb[pl.ds(j * 16, 16)] = v * 2 + c
                cp_r = pltpu.async_copy(tab_hbm.at[idxb], rows, sem_r)
            else:
                cp_r = pltpu.async_copy(tab_hbm.at[srcb], rows, sem_r)
            cp_s.wait()
            cp_d.wait()
            for j in range(CHUNK // 16):
                sv = sb[pl.ds(j * 16, 16)]
                dv = db[pl.ds(j * 16, 16)]
                t = sv + dv
                e = jnp.maximum(t, 0.2 * t)
                u = dv + g
                m = jnp.maximum(u, 0.2 * u)
                wb[pl.ds(j * 16, 16)] = jnp.exp(e - m)
            if do_den:
                @pl.when(c == 0)
                def _():
                    pltpu.sync_copy(wb, den_sh.at[dstb], add=True)
            cp_r.wait()

            def scale(r, acc):
                wr = wb[r]
                for q in range(D // 16):
                    rows[r, pl.ds(q * 16, 16)] = rows[r, pl.ds(q * 16, 16)] * wr
                return acc

            lax.fori_loop(0, CHUNK, scale, 0)
            pltpu.sync_copy(rows, acc_sh.at[dstb], add=True)
            return carry

        lax.fori_loop(0, n_chunks, chunk_body, 0)
        plsc.subcore_barrier()
        pltpu.sync_copy(acc_sh.at[pl.ds(base_rows, TILE_ROWS)],
                        acc_out.at[c, pl.ds(base_rows, TILE_ROWS)])
        if do_den:
            @pl.when(c == 0)
            def _():
                pltpu.sync_copy(den_sh.at[pl.ds(base_rows, TILE_ROWS)],
                                den_out.at[pl.ds(base_rows, TILE_ROWS)])

    return pl.kernel(body, out_type=out_type, mesh=mesh,
                     scratch_types=scratch)


def kernel(x, edge_index, batch, W1, a_src1, a_dst1, b1, W2, a_src2, a_dst2,
           b2, Wh1, bh1, Wh2, bh2):
    x = x.astype(jnp.float32)
    ei = edge_index.astype(jnp.int32)
    loop = jnp.arange(NN, dtype=jnp.int32)
    src = jnp.concatenate([ei[0], loop])
    dst = jnp.concatenate([ei[1], loop])
    epad = jnp.full((EPAD - src.shape[0],), NN, jnp.int32)
    srcp = jnp.concatenate([src, epad])
    dstp = jnp.concatenate([dst, epad])

    x16 = jnp.zeros((NPAD, 16), jnp.float32)
    x16 = x16.at[:NN, 0:3].set(x).at[:NN, 3].set(1.0)
    z16 = jnp.zeros((NPAD, 16), jnp.float32)
    z32 = jnp.zeros((NPAD, 32), jnp.float32)
    zd = jnp.zeros((NPAD,), jnp.float32)

    as1, ad1, g1 = _a1_call(x16, W1, a_src1.reshape(1, 64),
                            a_dst1.reshape(1, 64))

    sc1 = _sc_layer(16, split_edges=True, do_den=False, double_idx=False)
    acc1 = sc1(srcp, dstp, as1.reshape(NPAD), ad1.reshape(NPAD),
               g1.reshape(128), x16, z16, zd)
    if isinstance(acc1, (list, tuple)):
        acc1 = acc1[0]

    h2, as2, ad2, g2 = _b1_call(acc1, W1, b1.reshape(1, 64), W2,
                                a_src2.reshape(1, 64), a_dst2.reshape(1, 64))

    sc2 = _sc_layer(32, split_edges=False, do_den=True, double_idx=True)
    acc2, den2 = sc2(srcp, dstp, as2.reshape(NPAD), ad2.reshape(NPAD),
                     g2.reshape(128), h2.reshape(2 * NPAD, 32), z32, zd)

    batchp = jnp.concatenate([batch.astype(jnp.int32),
                              jnp.full((NPAD - NN,), 64, jnp.int32)])
    out = _c_call(acc2, den2.reshape(NPAD, 1), b2.reshape(1, 64),
                  batchp.reshape(NPAD, 1), Wh1, bh1.reshape(1, 128), Wh2,
                  bh2.reshape(1, 1))
    return out


# speedup vs baseline: 39.9663x; 39.9663x over previous
import jax
import jax.numpy as jnp
from jax import lax
from jax.experimental import pallas as pl
from jax.experimental.pallas import tpu as pltpu
from jax.experimental.pallas import tpu_sc as plsc

NN = 50000
NPAD = 51200
EPAD = 851968
CHUNK = 128
TILE_ROWS = NPAD // 16
NB = 8
BLK = NPAD // NB
NEG = -3.0e38
HP = lax.Precision.HIGHEST


def _dot(a, b, dims):
    return lax.dot_general(a, b, (dims, ((), ())),
                           preferred_element_type=jnp.float32, precision=HP)


def _a1_body(x_ref, w1_ref, asrc_ref, adst_ref, as_ref, ad_ref, g_ref, m_sc):
    i = pl.program_id(0)
    ab = jnp.concatenate([asrc_ref[...], adst_ref[...]], axis=0)
    cv = _dot(w1_ref[...], ab, ((1,), (1,)))
    xb = x_ref[...][:, 0:3]
    sd = _dot(xb, cv, ((1,), (0,)))
    as_ref[...] = sd[:, 0:1]
    ad_ref[...] = sd[:, 1:2]
    rows = i * BLK + lax.broadcasted_iota(jnp.int32, (BLK, 1), 0)
    bm = jnp.max(jnp.where(rows < NN, sd[:, 0:1], NEG))

    @pl.when(i == 0)
    def _():
        m_sc[0, 0] = NEG

    m_sc[0, 0] = jnp.maximum(m_sc[0, 0], bm)
    g_ref[...] = jnp.full((1, 128), m_sc[0, 0], jnp.float32)


def _a1_call(x16, W1, a_src1, a_dst1):
    return pl.pallas_call(
        _a1_body,
        grid=(NB,),
        in_specs=[
            pl.BlockSpec((BLK, 16), lambda i: (i, 0)),
            pl.BlockSpec((3, 64), lambda i: (0, 0)),
            pl.BlockSpec((1, 64), lambda i: (0, 0)),
            pl.BlockSpec((1, 64), lambda i: (0, 0)),
        ],
        out_specs=[
            pl.BlockSpec((BLK, 1), lambda i: (i, 0)),
            pl.BlockSpec((BLK, 1), lambda i: (i, 0)),
            pl.BlockSpec((1, 128), lambda i: (0, 0)),
        ],
        out_shape=[
            jax.ShapeDtypeStruct((NPAD, 1), jnp.float32),
            jax.ShapeDtypeStruct((NPAD, 1), jnp.float32),
            jax.ShapeDtypeStruct((1, 128), jnp.float32),
        ],
        scratch_shapes=[pltpu.SMEM((1, 1), jnp.float32)],
    )(x16, W1, a_src1, a_dst1)


def _b1_body(acc_ref, w1_ref, b1_ref, w2_ref, asrc_ref, adst_ref,
             h2_ref, as_ref, ad_ref, g_ref, m_sc):
    i = pl.program_id(0)
    a = acc_ref[...]
    agg = a[0] + a[1]
    xw = agg[:, 0:3] / (agg[:, 3:4] + 1e-16)
    h1 = jnp.maximum(_dot(xw, w1_ref[...], ((1,), (0,))) + b1_ref[...], 0.0)
    h2 = _dot(h1, w2_ref[...], ((1,), (0,)))
    h2_ref[...] = h2
    ab = jnp.concatenate([asrc_ref[...], adst_ref[...]], axis=0)
    sd = _dot(h2, ab, ((1,), (1,)))
    as_ref[...] = sd[:, 0:1]
    ad_ref[...] = sd[:, 1:2]
    rows = i * BLK + lax.broadcasted_iota(jnp.int32, (BLK, 1), 0)
    bm = jnp.max(jnp.where(rows < NN, sd[:, 0:1], NEG))

    @pl.when(i == 0)
    def _():
        m_sc[0, 0] = NEG

    m_sc[0, 0] = jnp.maximum(m_sc[0, 0], bm)
    g_ref[...] = jnp.full((1, 128), m_sc[0, 0], jnp.float32)


def _b1_call(acc1, W1, b1, W2, a_src2, a_dst2):
    return pl.pallas_call(
        _b1_body,
        grid=(NB,),
        in_specs=[
            pl.BlockSpec((2, BLK, 16), lambda i: (0, i, 0)),
            pl.BlockSpec((3, 64), lambda i: (0, 0)),
            pl.BlockSpec((1, 64), lambda i: (0, 0)),
            pl.BlockSpec((64, 64), lambda i: (0, 0)),
            pl.BlockSpec((1, 64), lambda i: (0, 0)),
            pl.BlockSpec((1, 64), lambda i: (0, 0)),
        ],
        out_specs=[
            pl.BlockSpec((BLK, 64), lambda i: (i, 0)),
            pl.BlockSpec((BLK, 1), lambda i: (i, 0)),
            pl.BlockSpec((BLK, 1), lambda i: (i, 0)),
            pl.BlockSpec((1, 128), lambda i: (0, 0)),
        ],
        out_shape=[
            jax.ShapeDtypeStruct((NPAD, 64), jnp.float32),
            jax.ShapeDtypeStruct((NPAD, 1), jnp.float32),
            jax.ShapeDtypeStruct((NPAD, 1), jnp.float32),
            jax.ShapeDtypeStruct((1, 128), jnp.float32),
        ],
        scratch_shapes=[pltpu.SMEM((1, 1), jnp.float32)],
    )(acc1, W1, b1, W2, a_src2, a_dst2)


def _c_body(acc_ref, dn_ref, b2_ref, batch_ref, wh1_ref, bh1_ref,
            wh2_ref, bh2_ref, out_ref, s_sc):
    i = pl.program_id(0)
    a = acc_ref[...]
    h = jnp.concatenate([a[0], a[1]], axis=1)
    h = jnp.maximum(h / (dn_ref[...] + 1e-16) + b2_ref[...], 0.0)
    oh = (batch_ref[...] == lax.broadcasted_iota(jnp.int32, (1, 64), 1))
    oh = oh.astype(jnp.float32)
    hx = jnp.concatenate([h, jnp.ones((BLK, 64), jnp.float32)], axis=1)
    ps = _dot(oh, hx, ((0,), (0,)))

    @pl.when(i == 0)
    def _():
        s_sc[...] = jnp.zeros((64, 128), jnp.float32)

    s_sc[...] += ps

    @pl.when(i == NB - 1)
    def _():
        sums = s_sc[:, 0:64]
        cnt = jnp.maximum(s_sc[:, 64:65], 1.0)
        pooled = sums / cnt
        t = jnp.maximum(_dot(pooled, wh1_ref[...], ((1,), (0,))) + bh1_ref[...],
                        0.0)
        out_ref[...] = _dot(t, wh2_ref[...], ((1,), (0,))) + bh2_ref[...]

    @pl.when(i < NB - 1)
    def _():
        out_ref[...] = jnp.zeros((64, 1), jnp.float32)


def _c_call(acc2, dn, b2, batchp, Wh1, bh1, Wh2, bh2):
    return pl.pallas_call(
        _c_body,
        grid=(NB,),
        in_specs=[
            pl.BlockSpec((2, BLK, 32), lambda i: (0, i, 0)),
            pl.BlockSpec((BLK, 1), lambda i: (i, 0)),
            pl.BlockSpec((1, 64), lambda i: (0, 0)),
            pl.BlockSpec((BLK, 1), lambda i: (i, 0)),
            pl.BlockSpec((64, 128), lambda i: (0, 0)),
            pl.BlockSpec((1, 128), lambda i: (0, 0)),
            pl.BlockSpec((128, 1), lambda i: (0, 0)),
            pl.BlockSpec((1, 1), lambda i: (0, 0)),
        ],
        out_specs=pl.BlockSpec((64, 1), lambda i: (0, 0)),
        out_shape=jax.ShapeDtypeStruct((64, 1), jnp.float32),
        scratch_shapes=[pltpu.VMEM((64, 128), jnp.float32)],
    )(acc2, dn, b2, batchp, Wh1, bh1, Wh2, bh2)


def _sc_layer(D, split_edges, do_den, double_idx):
    mesh = plsc.VectorSubcoreMesh(core_axis_name="c", subcore_axis_name="s",
                                  num_cores=2, num_subcores=16)
    out_type = [jax.ShapeDtypeStruct((2, NPAD, D), jnp.float32)]
    if do_den:
        out_type.append(jax.ShapeDtypeStruct((NPAD,), jnp.float32))
    scratch = [
        pltpu.VMEM_SHARED((NPAD, D), jnp.float32),
        pltpu.VMEM_SHARED((NPAD,), jnp.float32),
        pltpu.VMEM((2, CHUNK), jnp.int32),
        pltpu.VMEM((2, CHUNK), jnp.int32),
        pltpu.VMEM((2, CHUNK), jnp.int32),
        pltpu.VMEM((2, CHUNK), jnp.float32),
        pltpu.VMEM((2, CHUNK), jnp.float32),
        pltpu.VMEM((CHUNK,), jnp.float32),
        pltpu.VMEM((2, CHUNK, D), jnp.float32),
        pltpu.VMEM((128,), jnp.float32),
        pltpu.SemaphoreType.DMA,
        pltpu.SemaphoreType.DMA,
        pltpu.SemaphoreType.DMA,
        pltpu.SemaphoreType.DMA,
        pltpu.SemaphoreType.DMA,
        pltpu.SemaphoreType.DMA,
    ]

    if split_edges:
        n_chunks = EPAD // 32 // CHUNK
    else:
        n_chunks = EPAD // 16 // CHUNK

    def body(src_hbm, dst_hbm, as_hbm, ad_hbm, g_hbm, tab_hbm, z_hbm, zd_hbm,
             *refs):
        if do_den:
            acc_out, den_out = refs[0], refs[1]
            rest = refs[2:]
        else:
            acc_out = refs[0]
            rest = refs[1:]
        (acc_sh, den_sh, srcb, dstb, idxb, sb, db, wb, rows, gb,
         sem_s0, sem_d0, sem_r0, sem_s1, sem_d1, sem_r1) = rest
        bufs = [
            (srcb.at[0], dstb.at[0], idxb.at[0], sb.at[0], db.at[0],
             rows.at[0], sem_s0, sem_d0, sem_r0),
            (srcb.at[1], dstb.at[1], idxb.at[1], sb.at[1], db.at[1],
             rows.at[1], sem_s1, sem_d1, sem_r1),
        ]
        c = lax.axis_index("c")
        s = lax.axis_index("s")
        base_rows = s * TILE_ROWS
        pltpu.sync_copy(z_hbm.at[pl.ds(base_rows, TILE_ROWS)],
                        acc_sh.at[pl.ds(base_rows, TILE_ROWS)])
        if do_den:
            @pl.when(c == 0)
            def _():
                pltpu.sync_copy(zd_hbm.at[pl.ds(base_rows, TILE_ROWS)],
                                den_sh.at[pl.ds(base_rows, TILE_ROWS)])
        pltpu.sync_copy(g_hbm, gb)
        plsc.subcore_barrier()
        g = gb[pl.ds(0, 16)][0]
        if split_edges:
            tile_base = c * (EPAD // 2) + s * (EPAD // 32)
        else:
            tile_base = s * (EPAD // 16)

        def fire(ci, B):
            sB, dB, iB, _, _, rB, ss, sd, sr = bufs[B]
            base = tile_base + ci * CHUNK
            pltpu.sync_copy(src_hbm.at[pl.ds(base, CHUNK)], sB)
            pltpu.sync_copy(dst_hbm.at[pl.ds(base, CHUNK)], dB)
            pltpu.async_copy(as_hbm.at[sB], bufs[B][3], ss)
            pltpu.async_copy(ad_hbm.at[dB], bufs[B][4], sd)
            if double_idx:
                for j in range(CHUNK // 16):
                    v = sB[pl.ds(j * 16, 16)]
                    iB[pl.ds(j * 16, 16)] = v * 2 + c
                pltpu.async_copy(tab_hbm.at[iB], rB, sr)
            else:
                pltpu.async_copy(tab_hbm.at[sB], rB, sr)

        def drain(B):
            sB, dB, iB, sbB, dbB, rB, ss, sd, sr = bufs[B]
            pltpu.make_async_copy(as_hbm.at[sB], sbB, ss).wait()
            pltpu.make_async_copy(ad_hbm.at[dB], dbB, sd).wait()
            gi = iB if double_idx else sB
            pltpu.make_async_copy(tab_hbm.at[gi], rB, sr).wait()

        def process(B):
            sB, dB, iB, sbB, dbB, rB, ss, sd, sr = bufs[B]
            pltpu.make_async_copy(as_hbm.at[sB], sbB, ss).wait()
            pltpu.make_async_copy(ad_hbm.at[dB], dbB, sd).wait()
            for j in range(CHUNK // 16):
                sv = sbB[pl.ds(j * 16, 16)]
                dv = dbB[pl.ds(j * 16, 16)]
                t = sv + dv
                e = jnp.maximum(t, 0.2 * t)
                u = dv + g
                m = jnp.maximum(u, 0.2 * u)
                wb[pl.ds(j * 16, 16)] = jnp.exp(e - m)
            if do_den:
                @pl.when(c == 0)
                def _():
                    pltpu.sync_copy(wb, den_sh.at[dB], add=True)
            gi = iB if double_idx else sB
            pltpu.make_async_copy(tab_hbm.at[gi], rB, sr).wait()

            def scale(grp, acc):
                wv = wb[pl.ds(grp * 16, 16)]
                for r2 in range(16):
                    wr = wv[r2]
                    for q in range(D // 16):
                        rB[grp * 16 + r2, pl.ds(q * 16, 16)] = (
                            rB[grp * 16 + r2, pl.ds(q * 16, 16)] * wr)
                return acc

            lax.fori_loop(0, CHUNK // 16, scale, 0)
            pltpu.sync_copy(rB, acc_sh.at[dB], add=True)

        fire(0, 0)

        def body2(k, carry):
            c0 = 2 * k
            fire(jnp.minimum(c0 + 1, n_chunks - 1), 1)
            process(0)
            fire(jnp.minimum(c0 + 2, n_chunks - 1), 0)
            process(1)
            return carry

        lax.fori_loop(0, n_chunks // 2, body2, 0)
        drain(0)
        plsc.subcore_barrier()
        pltpu.sync_copy(acc_sh.at[pl.ds(base_rows, TILE_ROWS)],
                        acc_out.at[c, pl.ds(base_rows, TILE_ROWS)])
        if do_den:
            @pl.when(c == 0)
            def _():
                pltpu.sync_copy(den_sh.at[pl.ds(base_rows, TILE_ROWS)],
                                den_out.at[pl.ds(base_rows, TILE_ROWS)])

    return pl.kernel(body, out_type=out_type, mesh=mesh,
                     scratch_types=scratch,
                     compiler_params=pltpu.CompilerParams(
                         use_tc_tiling_on_sc=False))


def kernel(x, edge_index, batch, W1, a_src1, a_dst1, b1, W2, a_src2, a_dst2,
           b2, Wh1, bh1, Wh2, bh2):
    x = x.astype(jnp.float32)
    ei = edge_index.astype(jnp.int32)
    loop = jnp.arange(NN, dtype=jnp.int32)
    src = jnp.concatenate([ei[0], loop])
    dst = jnp.concatenate([ei[1], loop])
    epad = jnp.full((EPAD - src.shape[0],), NN, jnp.int32)
    srcp = jnp.concatenate([src, epad])
    dstp = jnp.concatenate([dst, epad])

    x16 = jnp.zeros((NPAD, 16), jnp.float32)
    x16 = x16.at[:NN, 0:3].set(x).at[:NN, 3].set(1.0)
    z16 = jnp.zeros((NPAD, 16), jnp.float32)
    z32 = jnp.zeros((NPAD, 32), jnp.float32)
    zd = jnp.zeros((NPAD,), jnp.float32)

    as1, ad1, g1 = _a1_call(x16, W1, a_src1.reshape(1, 64),
                            a_dst1.reshape(1, 64))

    sc1 = _sc_layer(16, split_edges=True, do_den=False, double_idx=False)
    acc1 = sc1(srcp, dstp, as1.reshape(NPAD), ad1.reshape(NPAD),
               g1.reshape(128), x16, z16, zd)
    if isinstance(acc1, (list, tuple)):
        acc1 = acc1[0]

    h2, as2, ad2, g2 = _b1_call(acc1, W1, b1.reshape(1, 64), W2,
                                a_src2.reshape(1, 64), a_dst2.reshape(1, 64))

    sc2 = _sc_layer(32, split_edges=False, do_den=True, double_idx=True)
    acc2, den2 = sc2(srcp, dstp, as2.reshape(NPAD), ad2.reshape(NPAD),
                     g2.reshape(128), h2.reshape(2 * NPAD, 32), z32, zd)

    batchp = jnp.concatenate([batch.astype(jnp.int32),
                              jnp.full((NPAD - NN,), 64, jnp.int32)])
    out = _c_call(acc2, den2.reshape(NPAD, 1), b2.reshape(1, 64),
                  batchp.reshape(NPAD, 1), Wh1, bh1.reshape(1, 128), Wh2,
                  bh2.reshape(1, 1))
    return out
